# Initial kernel scaffold; baseline (speedup 1.0000x reference)
import functools

import jax
import jax.numpy as jnp
from jax import lax
from jax.experimental import pallas as pl
from jax.experimental.pallas import tpu as pltpu
from jax.experimental.pallas import tpu_sc as plsc

NC = 2
NS = 16
NW = NC * NS
L = 16

CHUNK = 100
NG = 4
NO = 2
D = 32


@functools.lru_cache(maxsize=None)
def _build(n_chunks_total, vocab, seq, d):
    assert d == D
    chunks_per_w = n_chunks_total // NW
    n_groups = chunks_per_w // NG
    assert chunks_per_w % NG == 0
    mesh = plsc.VectorSubcoreMesh(core_axis_name="c", subcore_axis_name="s")

    @functools.partial(
        pl.kernel,
        mesh=mesh,
        out_type=jax.ShapeDtypeStruct((n_chunks_total * CHUNK, d), jnp.float32),
        scratch_types=(
            [pltpu.VMEM((chunks_per_w, CHUNK), jnp.int32),
             pltpu.VMEM((seq * d,), jnp.float32)]
            + [pltpu.VMEM((CHUNK, d), jnp.float32) for _ in range(NG)]
            + [pltpu.VMEM((CHUNK, d), jnp.float32) for _ in range(NO)]
            + [pltpu.SemaphoreType.DMA for _ in range(NG + NO)]
        ),
    )
    def emb(x_hbm, tok_hbm, pos_hbm, out_hbm, *scratch):
        idx_v = scratch[0]
        pos_v = scratch[1]
        gbufs = scratch[2:2 + NG]
        obufs = scratch[2 + NG:2 + NG + NO]
        gsems = scratch[2 + NG + NO:2 + NG + NO + NG]
        osems = scratch[2 + NG + NO + NG:]

        wid = lax.axis_index("s") * NC + lax.axis_index("c")
        base_chunk = wid * chunks_per_w

        pltpu.sync_copy(x_hbm.at[pl.ds(base_chunk, chunks_per_w)], idx_v)
        pltpu.sync_copy(pos_hbm, pos_v)

        def gather_start(j, slot):
            pltpu.make_async_copy(
                tok_hbm.at[idx_v.at[j]], gbufs[slot], gsems[slot]
            ).start()

        def gather_wait(j, slot):
            pltpu.make_async_copy(
                tok_hbm.at[idx_v.at[j]], gbufs[slot], gsems[slot]
            ).wait()

        def out_copy(j, slot):
            return pltpu.make_async_copy(
                obufs[slot],
                out_hbm.at[pl.ds((base_chunk + j) * CHUNK, CHUNK)],
                osems[slot],
            )

        for b in range(NG):
            gather_start(b, b)

        def group(g, _):
            j0 = g * NG
            for b in range(NG):
                j = j0 + b
                gslot = b
                oslot = b % NO
                pbase = (b % 2) * CHUNK
                gather_wait(j, gslot)
                @pl.when(j >= NO)
                def _():
                    out_copy(j - NO, oslot).wait()

                def addrow(i, _):
                    for h in range(d // L):
                        tok = gbufs[gslot][i, pl.ds(h * L, L)]
                        pos = pos_v[pl.ds((pbase + i) * d + h * L, L)]
                        obufs[oslot][i, pl.ds(h * L, L)] = tok + pos
                    return 0

                lax.fori_loop(0, CHUNK, addrow, 0, unroll=2)
                out_copy(j, oslot).start()
                @pl.when(j + NG < chunks_per_w)
                def _():
                    gather_start(j + NG, gslot)
            return 0

        lax.fori_loop(0, n_groups, group, 0)

        for b in range(NO):
            j = chunks_per_w - NO + b
            out_copy(j, j % NO).wait()

    return emb


def kernel(x, token_table, position_table):
    batch, seq = x.shape
    vocab, d = token_table.shape
    n_chunks_total = (batch * seq) // CHUNK
    x2 = x.reshape(n_chunks_total, CHUNK)
    pos_flat = position_table[:seq].reshape(-1)
    emb = _build(n_chunks_total, vocab, seq, d)
    out = emb(x2, token_table, pos_flat)
    return out.reshape(batch, seq, d)


# speedup vs baseline: 1.2360x; 1.2360x over previous
import functools

import jax
import jax.numpy as jnp
from jax import lax
from jax.experimental import pallas as pl
from jax.experimental.pallas import tpu as pltpu
from jax.experimental.pallas import tpu_sc as plsc

NC = 2
NS = 16
NW = NC * NS
L = 16

NG = 4
NO = 2
D = 32


@functools.lru_cache(maxsize=None)
def _build(batch, seq, vocab, d):
    assert d == D
    rows_per_w = batch // NW
    n_groups = rows_per_w // NG
    assert rows_per_w % NG == 0
    mesh = plsc.VectorSubcoreMesh(core_axis_name="c", subcore_axis_name="s")

    @functools.partial(
        pl.kernel,
        mesh=mesh,
        out_type=jax.ShapeDtypeStruct((batch, seq, d), jnp.float32),
        scratch_types=(
            [pltpu.VMEM((rows_per_w, seq), jnp.int32),
             pltpu.VMEM((seq * d,), jnp.float32)]
            + [pltpu.VMEM((seq, d), jnp.float32) for _ in range(NG)]
            + [pltpu.VMEM((seq, d), jnp.float32) for _ in range(NO)]
            + [pltpu.SemaphoreType.DMA for _ in range(NG + NO)]
        ),
        compiler_params=pltpu.CompilerParams(use_tc_tiling_on_sc=False),
    )
    def emb(x_hbm, tok_hbm, pos_hbm, out_hbm, *scratch):
        idx_v = scratch[0]
        pos_v = scratch[1]
        gbufs = scratch[2:2 + NG]
        obufs = scratch[2 + NG:2 + NG + NO]
        gsems = scratch[2 + NG + NO:2 + NG + NO + NG]
        osems = scratch[2 + NG + NO + NG:]

        wid = lax.axis_index("s") * NC + lax.axis_index("c")
        base_row = wid * rows_per_w

        pltpu.sync_copy(x_hbm.at[pl.ds(base_row, rows_per_w)], idx_v)
        pltpu.sync_copy(pos_hbm, pos_v)

        def gather_start(r, slot):
            pltpu.make_async_copy(
                tok_hbm.at[idx_v.at[r]], gbufs[slot], gsems[slot]
            ).start()

        def gather_wait(r, slot):
            pltpu.make_async_copy(
                tok_hbm.at[idx_v.at[r]], gbufs[slot], gsems[slot]
            ).wait()

        def out_copy(r, slot):
            return pltpu.make_async_copy(
                obufs[slot], out_hbm.at[base_row + r], osems[slot]
            )

        for b in range(NG):
            gather_start(b, b)

        def group(g, _):
            r0 = g * NG
            for b in range(NG):
                r = r0 + b
                oslot = b % NO
                gather_wait(r, b)
                @pl.when(r >= NO)
                def _():
                    out_copy(r - NO, oslot).wait()

                def addrow(i, _):
                    for h in range(d // L):
                        tok = gbufs[b][i, pl.ds(h * L, L)]
                        pos = pos_v[pl.ds(i * d + h * L, L)]
                        obufs[oslot][i, pl.ds(h * L, L)] = tok + pos
                    return 0

                lax.fori_loop(0, seq, addrow, 0, unroll=2)
                out_copy(r, oslot).start()
                @pl.when(r + NG < rows_per_w)
                def _():
                    gather_start(r + NG, b)
            return 0

        lax.fori_loop(0, n_groups, group, 0)

        for b in range(NO):
            r = rows_per_w - NO + b
            out_copy(r, r % NO).wait()

    return emb


def kernel(x, token_table, position_table):
    batch, seq = x.shape
    vocab, d = token_table.shape
    pos_flat = position_table[:seq].reshape(-1)
    emb = _build(batch, seq, vocab, d)
    return emb(x, token_table, pos_flat)
